# baseline (speedup 1.0000x reference)
import jax
import jax.numpy as jnp
from jax.experimental import pallas as pl
from jax.experimental.pallas import tpu as pltpu

LANE = 128


def _make_kernel(tb, m):
    def _kernel(x_ref, w_ref, b_ref, o_hbm, scratch, sems):
        c = pl.program_id(0)
        j = pl.program_id(1)
        p = j % 2
        d_out = o_hbm.shape[-1]

        @pl.when(j >= 2)
        def _():
            pltpu.make_async_copy(scratch.at[p], scratch.at[p], sems.at[p]).wait()

        acc = jnp.dot(x_ref[...], w_ref[...],
                      preferred_element_type=jnp.float32)
        scratch[p] = (acc + b_ref[...])[:, :d_out].astype(o_hbm.dtype)

        row0 = (c * m + j) * tb
        pltpu.make_async_copy(
            scratch.at[p], o_hbm.at[pl.ds(row0, tb), :], sems.at[p]).start()

        @pl.when(j == m - 1)
        def _():
            pltpu.make_async_copy(scratch.at[p], scratch.at[p], sems.at[p]).wait()
            if m >= 2:
                pltpu.make_async_copy(
                    scratch.at[1 - p], scratch.at[1 - p], sems.at[1 - p]).wait()
    return _kernel


def kernel(x, w, b):
    B, D_in = x.shape
    D_out = w.shape[0]

    w_t = jnp.zeros((D_in, LANE), x.dtype).at[:, :D_out].set(w.T.astype(x.dtype))
    b_p = jnp.zeros((1, LANE), x.dtype).at[0, :D_out].set(b.astype(x.dtype))

    tb = 8192
    n_tiles = B // tb
    m = n_tiles // 2

    out = pl.pallas_call(
        _make_kernel(tb, m),
        out_shape=jax.ShapeDtypeStruct((B, D_out), x.dtype),
        grid_spec=pltpu.PrefetchScalarGridSpec(
            num_scalar_prefetch=0,
            grid=(2, m),
            in_specs=[
                pl.BlockSpec((tb, D_in), lambda c, j: (c * m + j, 0)),
                pl.BlockSpec((D_in, LANE), lambda c, j: (0, 0)),
                pl.BlockSpec((1, LANE), lambda c, j: (0, 0)),
            ],
            out_specs=pl.BlockSpec(memory_space=pltpu.MemorySpace.HBM),
            scratch_shapes=[
                pltpu.VMEM((2, tb, D_out), jnp.float32),
                pltpu.SemaphoreType.DMA((2,)),
            ],
        ),
        compiler_params=pltpu.CompilerParams(
            dimension_semantics=("parallel", "arbitrary"),
        ),
    )(x, w_t, b_p)
    return out


# speedup vs baseline: 2.7515x; 2.7515x over previous
import jax
import jax.numpy as jnp
from jax.experimental import pallas as pl
from jax.experimental.pallas import tpu as pltpu

LANE = 128


def _linear_t_kernel(x_ref, w_ref, b_ref, o_ref):
    acc = jax.lax.dot_general(
        w_ref[...], x_ref[...],
        dimension_numbers=(((1,), (1,)), ((), ())),
        preferred_element_type=jnp.float32)
    o_ref[...] = (acc + b_ref[:, 0:1]).astype(o_ref.dtype)


def kernel(x, w, b):
    B, D_in = x.shape
    D_out = w.shape[0]

    b_p = jnp.zeros((D_out, LANE), x.dtype).at[:, 0].set(b.astype(x.dtype))

    tb = 8192
    n_tiles = B // tb

    out_t = pl.pallas_call(
        _linear_t_kernel,
        out_shape=jax.ShapeDtypeStruct((D_out, B), x.dtype),
        grid_spec=pltpu.PrefetchScalarGridSpec(
            num_scalar_prefetch=0,
            grid=(n_tiles,),
            in_specs=[
                pl.BlockSpec((tb, D_in), lambda i: (i, 0)),
                pl.BlockSpec((D_out, D_in), lambda i: (0, 0)),
                pl.BlockSpec((D_out, LANE), lambda i: (0, 0)),
            ],
            out_specs=pl.BlockSpec((D_out, tb), lambda i: (0, i)),
        ),
        compiler_params=pltpu.CompilerParams(
            dimension_semantics=("parallel",),
        ),
    )(x, w, b_p)
    return out_t.T


# speedup vs baseline: 2.9607x; 1.0760x over previous
import jax
import jax.numpy as jnp
from jax.experimental import pallas as pl
from jax.experimental.pallas import tpu as pltpu

LANE = 128


def _linear_t_kernel(x_ref, w_ref, b_ref, o_ref):
    acc = jax.lax.dot_general(
        w_ref[...], x_ref[...],
        dimension_numbers=(((1,), (1,)), ((), ())),
        preferred_element_type=jnp.float32)
    o_ref[...] = (acc + b_ref[:, 0:1]).astype(o_ref.dtype)


def kernel(x, w, b):
    B, D_in = x.shape
    D_out = w.shape[0]

    b_p = jnp.zeros((D_out, LANE), x.dtype).at[:, 0].set(b.astype(x.dtype))

    tb = 16384
    n_tiles = B // tb

    out_t = pl.pallas_call(
        _linear_t_kernel,
        out_shape=jax.ShapeDtypeStruct((D_out, B), x.dtype),
        grid_spec=pltpu.PrefetchScalarGridSpec(
            num_scalar_prefetch=0,
            grid=(n_tiles,),
            in_specs=[
                pl.BlockSpec((tb, D_in), lambda i: (i, 0)),
                pl.BlockSpec((D_out, D_in), lambda i: (0, 0)),
                pl.BlockSpec((D_out, LANE), lambda i: (0, 0)),
            ],
            out_specs=pl.BlockSpec((D_out, tb), lambda i: (0, i)),
        ),
        compiler_params=pltpu.CompilerParams(
            dimension_semantics=("parallel",),
        ),
    )(x, w, b_p)
    return out_t.T
